# baseline (speedup 1.0000x reference)
import functools

import jax
import jax.numpy as jnp
from jax import lax
from jax.experimental import pallas as pl
from jax.experimental.pallas import tpu as pltpu
from jax.experimental.pallas import tpu_sc as plsc

_NUM_FIELDS = 26
_VOCAB = 100000
_DIM = 128
_BATCH = 4096
_NUM_WORKERS = 32
_BPW = _BATCH // _NUM_WORKERS
_GRP = 2
_NGRP = _NUM_FIELDS // _GRP
_NBUF = 3


def _build_kernel():
    mesh = plsc.VectorSubcoreMesh(core_axis_name="c", subcore_axis_name="s")

    @functools.partial(
        pl.kernel,
        mesh=mesh,
        out_type=jax.ShapeDtypeStruct((_BATCH, _NUM_FIELDS, _DIM), jnp.float32),
        scratch_types=(
            [pltpu.VMEM((_NUM_FIELDS, _BPW), jnp.int32)]
            + [pltpu.VMEM((_BPW, _GRP, _DIM), jnp.float32)] * _NBUF
            + [pltpu.SemaphoreType.DMA] * (2 * _NBUF + 1)
        ),
    )
    def k(*rest):
        idxs_hbm = rest[:_NUM_FIELDS]
        ws = rest[_NUM_FIELDS:2 * _NUM_FIELDS]
        out_hbm = rest[2 * _NUM_FIELDS]
        scratch = rest[2 * _NUM_FIELDS + 1:]
        idx_v = scratch[0]
        bufs = scratch[1:1 + _NBUF]
        gsems = scratch[1 + _NBUF:1 + 2 * _NBUF]
        wsems = scratch[1 + 2 * _NBUF:1 + 3 * _NBUF]
        isem = scratch[1 + 3 * _NBUF]

        wid = lax.axis_index("s") * 2 + lax.axis_index("c")
        base = wid * _BPW
        icopies = [
            pltpu.make_async_copy(
                idxs_hbm[f].at[pl.ds(base, _BPW)], idx_v.at[f], isem
            )
            for f in range(_NUM_FIELDS)
        ]
        for c in icopies:
            c.start()
        for c in icopies:
            c.wait()

        gathers = []
        for g in range(_NGRP):
            slot = g % _NBUF
            gathers.append([
                pltpu.make_async_copy(
                    ws[_GRP * g + j].at[idx_v.at[_GRP * g + j]],
                    bufs[slot].at[:, j, :],
                    gsems[slot],
                )
                for j in range(_GRP)
            ])
        writes = []
        for g in range(_NGRP):
            slot = g % _NBUF
            writes.append(
                pltpu.make_async_copy(
                    bufs[slot],
                    out_hbm.at[pl.ds(base, _BPW), pl.ds(_GRP * g, _GRP), :],
                    wsems[slot],
                )
            )

        def start_group(g):
            for c in gathers[g]:
                c.start()

        def wait_group(g):
            for c in gathers[g]:
                c.wait()

        for g in range(min(_NBUF, _NGRP)):
            start_group(g)
        for g in range(_NGRP):
            wait_group(g)
            writes[g].start()
            if g + _NBUF < _NGRP:
                writes[g].wait()
                start_group(g + _NBUF)
        for g in range(max(0, _NGRP - _NBUF), _NGRP):
            writes[g].wait()

    return k


_kernel_call = _build_kernel()


@jax.jit
def kernel(f0, f1, f2, f3, f4, f5, f6, f7, f8, f9, f10, f11, f12, f13, f14,
           f15, f16, f17, f18, f19, f20, f21, f22, f23, f24, f25,
           W0, W1, W2, W3, W4, W5, W6, W7, W8, W9, W10, W11, W12, W13, W14,
           W15, W16, W17, W18, W19, W20, W21, W22, W23, W24, W25):
    idxs = [f0, f1, f2, f3, f4, f5, f6, f7, f8, f9, f10, f11, f12, f13, f14,
            f15, f16, f17, f18, f19, f20, f21, f22, f23, f24, f25]
    ws = [W0, W1, W2, W3, W4, W5, W6, W7, W8, W9, W10, W11, W12, W13, W14,
          W15, W16, W17, W18, W19, W20, W21, W22, W23, W24, W25]
    idxs = [i.astype(jnp.int32) for i in idxs]
    out = _kernel_call(*idxs, *ws)
    return out.reshape(_BATCH, _NUM_FIELDS * _DIM)


# speedup vs baseline: 2.7937x; 2.7937x over previous
import functools

import jax
import jax.numpy as jnp
from jax import lax
from jax.experimental import pallas as pl
from jax.experimental.pallas import tpu as pltpu
from jax.experimental.pallas import tpu_sc as plsc

_NUM_FIELDS = 26
_VOCAB = 100000
_DIM = 128
_BATCH = 4096
_NUM_WORKERS = 32
_BPW = _BATCH // _NUM_WORKERS
_NBUF = 7


def _build_kernel():
    mesh = plsc.VectorSubcoreMesh(core_axis_name="c", subcore_axis_name="s")

    @functools.partial(
        pl.kernel,
        mesh=mesh,
        out_type=jax.ShapeDtypeStruct((_BATCH, _NUM_FIELDS * _DIM), jnp.float32),
        scratch_types=(
            [pltpu.VMEM((_NUM_FIELDS, _BPW), jnp.int32)]
            + [pltpu.VMEM((_BPW, _DIM), jnp.float32)] * _NBUF
            + [pltpu.SemaphoreType.DMA] * (2 * _NBUF + 1)
        ),
    )
    def k(*rest):
        idxs_hbm = rest[:_NUM_FIELDS]
        ws = rest[_NUM_FIELDS:2 * _NUM_FIELDS]
        out_hbm = rest[2 * _NUM_FIELDS]
        scratch = rest[2 * _NUM_FIELDS + 1:]
        idx_v = scratch[0]
        bufs = scratch[1:1 + _NBUF]
        gsems = scratch[1 + _NBUF:1 + 2 * _NBUF]
        wsems = scratch[1 + 2 * _NBUF:1 + 3 * _NBUF]
        isem = scratch[1 + 3 * _NBUF]

        wid = lax.axis_index("s") * 2 + lax.axis_index("c")
        base = wid * _BPW
        icopies = [
            pltpu.make_async_copy(
                idxs_hbm[f].at[pl.ds(base, _BPW)], idx_v.at[f], isem
            )
            for f in range(_NUM_FIELDS)
        ]
        for c in icopies:
            c.start()
        for c in icopies:
            c.wait()

        nbuf = _NBUF
        gathers = []
        for f in range(_NUM_FIELDS):
            gathers.append(
                pltpu.make_async_copy(
                    ws[f].at[idx_v.at[f]], bufs[f % nbuf], gsems[f % nbuf]
                )
            )
        writes = []
        for f in range(_NUM_FIELDS):
            writes.append(
                pltpu.make_async_copy(
                    bufs[f % nbuf],
                    out_hbm.at[pl.ds(base, _BPW), pl.ds(f * _DIM, _DIM)],
                    wsems[f % nbuf],
                )
            )

        lag = 3
        for f in range(min(nbuf, _NUM_FIELDS)):
            gathers[f].start()
        waited = set()
        for f in range(_NUM_FIELDS):
            gathers[f].wait()
            writes[f].start()
            r = f - lag
            if r >= 0 and r + nbuf < _NUM_FIELDS:
                writes[r].wait()
                waited.add(r)
                gathers[r + nbuf].start()
        for f in range(_NUM_FIELDS):
            if f not in waited:
                writes[f].wait()

    return k


_kernel_call = _build_kernel()


@jax.jit
def kernel(f0, f1, f2, f3, f4, f5, f6, f7, f8, f9, f10, f11, f12, f13, f14,
           f15, f16, f17, f18, f19, f20, f21, f22, f23, f24, f25,
           W0, W1, W2, W3, W4, W5, W6, W7, W8, W9, W10, W11, W12, W13, W14,
           W15, W16, W17, W18, W19, W20, W21, W22, W23, W24, W25):
    idxs = [f0, f1, f2, f3, f4, f5, f6, f7, f8, f9, f10, f11, f12, f13, f14,
            f15, f16, f17, f18, f19, f20, f21, f22, f23, f24, f25]
    ws = [W0, W1, W2, W3, W4, W5, W6, W7, W8, W9, W10, W11, W12, W13, W14,
          W15, W16, W17, W18, W19, W20, W21, W22, W23, W24, W25]
    idxs = [i.astype(jnp.int32) for i in idxs]
    return _kernel_call(*idxs, *ws)
